# baseline (speedup 1.0000x reference)
import dataclasses
import functools

import jax
import jax.numpy as jnp
from jax import lax
from jax.experimental import pallas as pl
from jax.experimental.pallas import tpu as pltpu
from jax.experimental.pallas import tpu_sc as plsc

B, L, D = 16, 2048, 300
NC, NS = 2, 16
NW = NC * NS
LANES = 16
CR = 64
CHUNK = CR * D
FOLD = 1200
NVR = FOLD // LANES
PAIRS = 2 * B


def _sc_partial_sums(prem2d, lens_p, hyp2d, lens_h):
    mesh = plsc.VectorSubcoreMesh(
        core_axis_name="c", subcore_axis_name="s",
        num_cores=NC, num_subcores=NS)
    cp = pltpu.CompilerParams()
    if "needs_layout_passes" in pltpu.CompilerParams.__dataclass_fields__:
        cp = dataclasses.replace(cp, needs_layout_passes=False)

    @functools.partial(
        pl.kernel,
        compiler_params=cp,
        out_type=jax.ShapeDtypeStruct((NW, PAIRS * FOLD), jnp.float32),
        mesh=mesh,
        scratch_types=[
            pltpu.VMEM((CHUNK,), jnp.float32),
            pltpu.VMEM((CHUNK,), jnp.float32),
            pltpu.VMEM((PAIRS * FOLD,), jnp.float32),
            pltpu.VMEM((LANES,), jnp.int32),
            pltpu.VMEM((LANES,), jnp.int32),
            pltpu.SemaphoreType.DMA,
            pltpu.SemaphoreType.DMA,
        ],
    )
    def k(prem_hbm, lenp_hbm, hyp_hbm, lenh_hbm, out_hbm,
          bufp, bufh, acc, lpv, lhv, semp, semh):
        wid = lax.axis_index("s") * NC + lax.axis_index("c")
        pltpu.sync_copy(lenp_hbm, lpv)
        pltpu.sync_copy(lenh_hbm, lhv)
        iota = lax.iota(jnp.int32, LANES)
        zero = jnp.zeros((LANES,), jnp.float32)

        @pl.loop(0, PAIRS * NVR)
        def _(i):
            acc[pl.ds(i * LANES, LANES)] = zero

        lpvec = lpv[...]
        lhvec = lhv[...]

        def lane(vec, s):
            return lax.reduce_max(jnp.where(iota == s, vec, 0), axes=(0,))

        def chunk_valid(length, p):
            d = lax.rem(wid + NW - p, NW)
            return d, jnp.clip(length - d * CR, 0, CR) * D

        def accumulate(buf, slot, nvalid):
            ngroups = lax.div(nvalid + (FOLD - 1), FOLD)

            @pl.loop(0, ngroups)
            def _(g):
                tbase = nvalid - g * FOLD
                for a in range(NVR):
                    x = buf[pl.ds(g * FOLD + a * LANES, LANES)]
                    m = iota < (tbase - a * LANES)
                    plsc.addupdate(
                        acc.at[pl.ds(slot * FOLD + a * LANES, LANES)],
                        jnp.where(m, x, 0.0))

        @pl.loop(0, B)
        def _(s):
            len_p = lane(lpvec, s)
            len_h = lane(lhvec, s)
            dp, nvp = chunk_valid(len_p, s)
            dh, nvh = chunk_valid(len_h, s + B)

            @pl.when(nvp > 0)
            def _():
                pltpu.async_copy(
                    prem_hbm.at[s, pl.ds(dp * CHUNK, CHUNK)], bufp, semp)

            @pl.when(nvh > 0)
            def _():
                pltpu.async_copy(
                    hyp_hbm.at[s, pl.ds(dh * CHUNK, CHUNK)], bufh, semh)

            @pl.when(nvp > 0)
            def _():
                pltpu.make_async_copy(
                    prem_hbm.at[s, pl.ds(dp * CHUNK, CHUNK)], bufp, semp).wait()
                accumulate(bufp, s, nvp)

            @pl.when(nvh > 0)
            def _():
                pltpu.make_async_copy(
                    hyp_hbm.at[s, pl.ds(dh * CHUNK, CHUNK)], bufh, semh).wait()
                accumulate(bufh, s + B, nvh)

        pltpu.sync_copy(acc, out_hbm.at[wid])

    return k(prem2d, lens_p, hyp2d, lens_h)


def _finale(partials, lengths_p, lengths_h):
    part4 = partials.reshape(NW, PAIRS, FOLD // D, D)

    def body(part_ref, lp_ref, lh_ref, out_ref):
        sall = jnp.sum(part_ref[...], axis=(0, 2))
        p = sall[0:B, :] / lp_ref[...]
        h = sall[B:2 * B, :] / lh_ref[...]
        out_ref[:, 0, :] = p
        out_ref[:, 1, :] = h
        out_ref[:, 2, :] = jnp.abs(p - h)
        out_ref[:, 3, :] = p * h

    out = pl.pallas_call(
        body,
        out_shape=jax.ShapeDtypeStruct((B, 4, D), jnp.float32),
    )(part4,
      lengths_p.astype(jnp.float32).reshape(B, 1),
      lengths_h.astype(jnp.float32).reshape(B, 1))
    return out.reshape(B, 4 * D)


def kernel(premises, lengths_premises, hypothesis, lengths_hypothesis):
    prem2d = premises.reshape(B, L * D)
    hyp2d = hypothesis.reshape(B, L * D)
    lp = lengths_premises.astype(jnp.int32)
    lh = lengths_hypothesis.astype(jnp.int32)
    partials = _sc_partial_sums(prem2d, lp, hyp2d, lh)
    return _finale(partials, lengths_premises, lengths_hypothesis)


# speedup vs baseline: 1.6808x; 1.6808x over previous
import dataclasses
import functools

import jax
import jax.numpy as jnp
from jax import lax
from jax.experimental import pallas as pl
from jax.experimental.pallas import tpu as pltpu
from jax.experimental.pallas import tpu_sc as plsc

B, L, D = 16, 2048, 300
NC, NS = 2, 16
NW = NC * NS
LANES = 16
CR = 64
NFULL = D // LANES
TAIL0 = D - LANES
ACCW = (NFULL + 1) * LANES
PAIRS = 2 * B


def _sc_partial_sums(prem, lens_p, hyp, lens_h):
    mesh = plsc.VectorSubcoreMesh(
        core_axis_name="c", subcore_axis_name="s",
        num_cores=NC, num_subcores=NS)
    cp = pltpu.CompilerParams()
    if "needs_layout_passes" in pltpu.CompilerParams.__dataclass_fields__:
        cp = dataclasses.replace(cp, needs_layout_passes=False)

    @functools.partial(
        pl.kernel,
        compiler_params=cp,
        out_type=jax.ShapeDtypeStruct((NW, PAIRS * ACCW), jnp.float32),
        mesh=mesh,
        scratch_types=[
            pltpu.VMEM((CR, D), jnp.float32),
            pltpu.VMEM((CR, D), jnp.float32),
            pltpu.VMEM((PAIRS * ACCW,), jnp.float32),
            pltpu.VMEM((LANES,), jnp.int32),
            pltpu.VMEM((LANES,), jnp.int32),
            pltpu.SemaphoreType.DMA,
            pltpu.SemaphoreType.DMA,
        ],
    )
    def k(prem_hbm, lenp_hbm, hyp_hbm, lenh_hbm, out_hbm,
          bufp, bufh, acc, lpv, lhv, semp, semh):
        wid = lax.axis_index("s") * NC + lax.axis_index("c")
        pltpu.sync_copy(lenp_hbm, lpv)
        pltpu.sync_copy(lenh_hbm, lhv)
        iota = lax.iota(jnp.int32, LANES)
        tailmask = iota >= (LANES - (D - NFULL * LANES))
        zero = jnp.zeros((LANES,), jnp.float32)

        @pl.loop(0, PAIRS * (NFULL + 1))
        def _(i):
            acc[pl.ds(i * LANES, LANES)] = zero

        lpvec = lpv[...]
        lhvec = lhv[...]

        def lane(vec, s):
            return lax.reduce_max(jnp.where(iota == s, vec, 0), axes=(0,))

        def chunk_rows(length, p):
            d = lax.rem(wid + NW - p, NW)
            return d, jnp.clip(length - d * CR, 0, CR)

        def accumulate(buf, slot, vrows):
            base = slot * ACCW

            @pl.loop(0, vrows)
            def _(r):
                for a in range(NFULL):
                    x = buf[r, pl.ds(a * LANES, LANES)]
                    plsc.addupdate(acc.at[pl.ds(base + a * LANES, LANES)], x)
                x = buf[r, pl.ds(TAIL0, LANES)]
                plsc.addupdate(acc.at[pl.ds(base + NFULL * LANES, LANES)],
                               jnp.where(tailmask, x, 0.0))

        @pl.loop(0, B)
        def _(s):
            len_p = lane(lpvec, s)
            len_h = lane(lhvec, s)
            dp, nrp = chunk_rows(len_p, s)
            dh, nrh = chunk_rows(len_h, s + B)

            @pl.when(nrp > 0)
            def _():
                pltpu.async_copy(
                    prem_hbm.at[s, pl.ds(dp * CR, CR), :], bufp, semp)

            @pl.when(nrh > 0)
            def _():
                pltpu.async_copy(
                    hyp_hbm.at[s, pl.ds(dh * CR, CR), :], bufh, semh)

            @pl.when(nrp > 0)
            def _():
                pltpu.make_async_copy(
                    prem_hbm.at[s, pl.ds(dp * CR, CR), :], bufp, semp).wait()
                accumulate(bufp, s, nrp)

            @pl.when(nrh > 0)
            def _():
                pltpu.make_async_copy(
                    hyp_hbm.at[s, pl.ds(dh * CR, CR), :], bufh, semh).wait()
                accumulate(bufh, s + B, nrh)

        pltpu.sync_copy(acc, out_hbm.at[wid])

    return k(prem, lens_p, hyp, lens_h)


def _finale(partials, lengths_p, lengths_h):
    def body(part_ref, lp_ref, lh_ref, out_ref):
        sall = jnp.sum(part_ref[...], axis=0)
        s300 = jnp.concatenate(
            [sall[:, 0:NFULL * LANES],
             sall[:, NFULL * LANES + (ACCW - D):ACCW]], axis=1)
        p = s300[0:B, :] / lp_ref[...]
        h = s300[B:2 * B, :] / lh_ref[...]
        out_ref[:, 0, :] = p
        out_ref[:, 1, :] = h
        out_ref[:, 2, :] = jnp.abs(p - h)
        out_ref[:, 3, :] = p * h

    out = pl.pallas_call(
        body,
        out_shape=jax.ShapeDtypeStruct((B, 4, D), jnp.float32),
    )(partials.reshape(NW, PAIRS, ACCW),
      lengths_p.astype(jnp.float32).reshape(B, 1),
      lengths_h.astype(jnp.float32).reshape(B, 1))
    return out.reshape(B, 4 * D)


def kernel(premises, lengths_premises, hypothesis, lengths_hypothesis):
    lp = lengths_premises.astype(jnp.int32)
    lh = lengths_hypothesis.astype(jnp.int32)
    partials = _sc_partial_sums(premises, lp, hypothesis, lh)
    return _finale(partials, lengths_premises, lengths_hypothesis)


# speedup vs baseline: 2.4833x; 1.4775x over previous
import dataclasses
import functools

import jax
import jax.numpy as jnp
from jax import lax
from jax.experimental import pallas as pl
from jax.experimental.pallas import tpu as pltpu
from jax.experimental.pallas import tpu_sc as plsc

B, L, D = 16, 2048, 300
NC, NS = 2, 16
NW = NC * NS
LANES = 16
CR = 64
NFULL = D // LANES
TAIL0 = D - LANES
ACCW = (NFULL + 1) * LANES
PAIRS = 2 * B


def _sc_partial_sums(prem, lens_p, hyp, lens_h):
    mesh = plsc.VectorSubcoreMesh(
        core_axis_name="c", subcore_axis_name="s",
        num_cores=NC, num_subcores=NS)
    cp = pltpu.CompilerParams(use_tc_tiling_on_sc=True)
    if "needs_layout_passes" in pltpu.CompilerParams.__dataclass_fields__:
        cp = dataclasses.replace(cp, needs_layout_passes=False)

    @functools.partial(
        pl.kernel,
        compiler_params=cp,
        out_type=jax.ShapeDtypeStruct((NW, PAIRS * ACCW), jnp.float32),
        mesh=mesh,
        scratch_types=[
            pltpu.VMEM((CR, D), jnp.float32),
            pltpu.VMEM((CR, D), jnp.float32),
            pltpu.VMEM((CR, D), jnp.float32),
            pltpu.VMEM((CR, D), jnp.float32),
            pltpu.VMEM((PAIRS * ACCW,), jnp.float32),
            pltpu.VMEM((LANES,), jnp.int32),
            pltpu.VMEM((LANES,), jnp.int32),
            pltpu.SemaphoreType.DMA,
            pltpu.SemaphoreType.DMA,
            pltpu.SemaphoreType.DMA,
            pltpu.SemaphoreType.DMA,
        ],
    )
    def k(prem_hbm, lenp_hbm, hyp_hbm, lenh_hbm, out_hbm,
          bufp0, bufp1, bufh0, bufh1, acc, lpv, lhv,
          semp0, semp1, semh0, semh1):
        wid = lax.axis_index("s") * NC + lax.axis_index("c")
        pltpu.sync_copy(lenp_hbm, lpv)
        pltpu.sync_copy(lenh_hbm, lhv)
        iota = lax.iota(jnp.int32, LANES)
        tailmask = iota >= (LANES - (D - NFULL * LANES))

        lpvec = lpv[...]
        lhvec = lhv[...]

        def lane(vec, s):
            return lax.reduce_max(jnp.where(iota == s, vec, 0), axes=(0,))

        def seq_info(s):
            def chunk_rows(length, p):
                d = lax.rem(wid + NW - p, NW)
                return d, jnp.clip(length - d * CR, 0, CR)
            dp, nrp = chunk_rows(lane(lpvec, s), s)
            dh, nrh = chunk_rows(lane(lhvec, s), s + B)
            return dp, nrp, dh, nrh

        def start_dmas(s, bufp, bufh, semp, semh):
            dp, nrp, dh, nrh = seq_info(s)

            @pl.when(nrp > 0)
            def _():
                pltpu.async_copy(
                    prem_hbm.at[s, pl.ds(dp * CR, CR), :], bufp, semp)

            @pl.when(nrh > 0)
            def _():
                pltpu.async_copy(
                    hyp_hbm.at[s, pl.ds(dh * CR, CR), :], bufh, semh)

        def accumulate(buf, slot, vrows):
            init = tuple(jnp.zeros((LANES,), jnp.float32)
                         for _ in range(NFULL + 1))

            def row_add(r, regs):
                new = [regs[a] + buf[r, pl.ds(a * LANES, LANES)]
                       for a in range(NFULL)]
                tail = buf[r, pl.ds(TAIL0, LANES)]
                new.append(regs[NFULL] + jnp.where(tailmask, tail, 0.0))
                return tuple(new)

            final = pl.loop(0, vrows, init_carry=init)(row_add)

            base = slot * ACCW
            for a in range(NFULL + 1):
                acc[pl.ds(base + a * LANES, LANES)] = final[a]

        def consume(s, bufp, bufh, semp, semh):
            dp, nrp, dh, nrh = seq_info(s)

            @pl.when(nrp > 0)
            def _():
                pltpu.make_async_copy(
                    prem_hbm.at[s, pl.ds(dp * CR, CR), :], bufp, semp).wait()
            accumulate(bufp, s, nrp)

            @pl.when(nrh > 0)
            def _():
                pltpu.make_async_copy(
                    hyp_hbm.at[s, pl.ds(dh * CR, CR), :], bufh, semh).wait()
            accumulate(bufh, s + B, nrh)

        start_dmas(0, bufp0, bufh0, semp0, semh0)

        @pl.loop(0, B // 2)
        def _(t):
            s0 = 2 * t
            start_dmas(s0 + 1, bufp1, bufh1, semp1, semh1)
            consume(s0, bufp0, bufh0, semp0, semh0)

            @pl.when(s0 + 2 < B)
            def _():
                start_dmas(s0 + 2, bufp0, bufh0, semp0, semh0)
            consume(s0 + 1, bufp1, bufh1, semp1, semh1)

        pltpu.sync_copy(acc, out_hbm.at[wid])

    return k(prem, lens_p, hyp, lens_h)


def _finale(partials, lengths_p, lengths_h):
    def body(part_ref, lp_ref, lh_ref, out_ref):
        sall = jnp.sum(part_ref[...], axis=0)
        s300 = jnp.concatenate(
            [sall[:, 0:NFULL * LANES],
             sall[:, NFULL * LANES + (ACCW - D):ACCW]], axis=1)
        p = s300[0:B, :] / lp_ref[...]
        h = s300[B:2 * B, :] / lh_ref[...]
        out_ref[:, 0, :] = p
        out_ref[:, 1, :] = h
        out_ref[:, 2, :] = jnp.abs(p - h)
        out_ref[:, 3, :] = p * h

    out = pl.pallas_call(
        body,
        out_shape=jax.ShapeDtypeStruct((B, 4, D), jnp.float32),
    )(partials.reshape(NW, PAIRS, ACCW),
      lengths_p.astype(jnp.float32).reshape(B, 1),
      lengths_h.astype(jnp.float32).reshape(B, 1))
    return out.reshape(B, 4 * D)


def kernel(premises, lengths_premises, hypothesis, lengths_hypothesis):
    lp = lengths_premises.astype(jnp.int32)
    lh = lengths_hypothesis.astype(jnp.int32)
    partials = _sc_partial_sums(premises, lp, hypothesis, lh)
    return _finale(partials, lengths_premises, lengths_hypothesis)


# speedup vs baseline: 3.5898x; 1.4456x over previous
import dataclasses
import functools

import jax
import jax.numpy as jnp
from jax import lax
from jax.experimental import pallas as pl
from jax.experimental.pallas import tpu as pltpu
from jax.experimental.pallas import tpu_sc as plsc

B, L, D = 16, 2048, 300
NC, NS = 2, 16
NW = NC * NS
LANES = 16
NFG = (D + LANES - 1) // LANES
DPAD = NFG * LANES
PAIRS = 2 * B

KCOLS = 1280
CC = 256
TCR = 128
NTCH = (L - KCOLS) // TCR
CVR = TCR // LANES


def _tc_dense_sums(prem_t, hyp_t, lens_p2, lens_h2):
    def body(lp_ref, lh_ref, p_ref, h_ref, pout_ref, hout_ref):
        j = pl.program_id(0)
        ci = lax.broadcasted_iota(jnp.int32, (1, B, CC), 2) + j * CC
        mp = (ci < lp_ref[...].reshape(1, B, 1)).astype(jnp.float32)
        mh = (ci < lh_ref[...].reshape(1, B, 1)).astype(jnp.float32)
        ps = jnp.sum(p_ref[...] * mp, axis=2)
        hs = jnp.sum(h_ref[...] * mh, axis=2)

        @pl.when(j == 0)
        def _():
            pout_ref[...] = ps
            hout_ref[...] = hs

        @pl.when(j > 0)
        def _():
            pout_ref[...] += ps
            hout_ref[...] += hs

    return pl.pallas_call(
        body,
        grid=(KCOLS // CC,),
        in_specs=[
            pl.BlockSpec((1, B), lambda j: (0, 0)),
            pl.BlockSpec((1, B), lambda j: (0, 0)),
            pl.BlockSpec((D, B, CC), lambda j: (0, 0, j)),
            pl.BlockSpec((D, B, CC), lambda j: (0, 0, j)),
        ],
        out_specs=[
            pl.BlockSpec((D, B), lambda j: (0, 0)),
            pl.BlockSpec((D, B), lambda j: (0, 0)),
        ],
        out_shape=[jax.ShapeDtypeStruct((D, B), jnp.float32)] * 2,
    )(lens_p2, lens_h2, prem_t, hyp_t)


def _sc_tail_sums(prem_t, lens_p, hyp_t, lens_h):
    mesh = plsc.VectorSubcoreMesh(
        core_axis_name="c", subcore_axis_name="s",
        num_cores=NC, num_subcores=NS)
    cp = pltpu.CompilerParams()
    if "needs_layout_passes" in pltpu.CompilerParams.__dataclass_fields__:
        cp = dataclasses.replace(cp, needs_layout_passes=False)

    @functools.partial(
        pl.kernel,
        compiler_params=cp,
        out_type=jax.ShapeDtypeStruct((PAIRS, NTCH, DPAD * LANES),
                                      jnp.float32),
        mesh=mesh,
        scratch_types=[
            pltpu.VMEM((DPAD, TCR), jnp.float32),
            pltpu.VMEM((DPAD, TCR), jnp.float32),
            pltpu.VMEM((DPAD * LANES,), jnp.float32),
            pltpu.VMEM((DPAD * LANES,), jnp.float32),
            pltpu.VMEM((LANES,), jnp.int32),
            pltpu.VMEM((LANES,), jnp.int32),
            pltpu.SemaphoreType.DMA,
            pltpu.SemaphoreType.DMA,
        ],
    )
    def k(prem_hbm, lenp_hbm, hyp_hbm, lenh_hbm, out_hbm,
          buf0, buf1, acc0, acc1, lpv, lhv, sem0, sem1):
        wid = lax.axis_index("s") * NC + lax.axis_index("c")
        pltpu.sync_copy(lenp_hbm, lpv)
        pltpu.sync_copy(lenh_hbm, lhv)
        iota = lax.iota(jnp.int32, LANES)
        fzero = jnp.zeros((LANES,), jnp.float32)

        lpvec = lpv[...]
        lhvec = lhv[...]

        def lane(vec, s):
            return lax.reduce_max(jnp.where(iota == s, vec, 0), axes=(0,))

        def slot_info(d):
            p = lax.rem(wid + NW - d, NW)
            pm = lax.rem(p, B)
            length = jnp.where(p < B, lane(lpvec, pm), lane(lhvec, pm))
            vcols = jnp.clip(length - (KCOLS + d * TCR), 0, TCR)
            return p, pm, vcols

        def start_dma(d, buf, sem):
            p, pm, vcols = slot_info(d)
            c0 = KCOLS + d * TCR

            @pl.when((vcols > 0) & (p < B))
            def _():
                pltpu.async_copy(
                    prem_hbm.at[:, pm, pl.ds(c0, TCR)],
                    buf.at[pl.ds(0, D), :], sem)

            @pl.when((vcols > 0) & (p >= B))
            def _():
                pltpu.async_copy(
                    hyp_hbm.at[:, pm, pl.ds(c0, TCR)],
                    buf.at[pl.ds(0, D), :], sem)

        def consume(d, buf, acc, sem):
            p, pm, vcols = slot_info(d)
            c0 = KCOLS + d * TCR

            @pl.when((vcols > 0) & (p < B))
            def _():
                pltpu.make_async_copy(
                    prem_hbm.at[:, pm, pl.ds(c0, TCR)],
                    buf.at[pl.ds(0, D), :], sem).wait()

            @pl.when((vcols > 0) & (p >= B))
            def _():
                pltpu.make_async_copy(
                    hyp_hbm.at[:, pm, pl.ds(c0, TCR)],
                    buf.at[pl.ds(0, D), :], sem).wait()

            @pl.when(vcols > 0)
            def _():
                masks = [(c * LANES + iota) < vcols for c in range(CVR)]

                @pl.loop(0, DPAD, step=2)
                def _(f0):
                    for f in (f0, f0 + 1):
                        v = fzero
                        for c in range(CVR):
                            x = buf[f, pl.ds(c * LANES, LANES)]
                            v = v + jnp.where(masks[c], x, 0.0)
                        acc[pl.ds(f * LANES, LANES)] = v

            @pl.when(vcols == 0)
            def _():
                @pl.loop(0, DPAD)
                def _(f):
                    acc[pl.ds(f * LANES, LANES)] = fzero

            pltpu.sync_copy(acc, out_hbm.at[p, d])

        bufs = (buf0, buf1)
        accs = (acc0, acc1)
        sems = (sem0, sem1)
        start_dma(0, bufs[0], sems[0])
        for d in range(NTCH):
            if d + 1 < NTCH:
                start_dma(d + 1, bufs[(d + 1) % 2], sems[(d + 1) % 2])
            consume(d, bufs[d % 2], accs[d % 2], sems[d % 2])

    out = k(prem_t, lens_p, hyp_t, lens_h)
    return out


def _finale(tcp, tch, tails, lengths_p, lengths_h):
    def body(tcp_ref, tch_ref, tail_ref, lp_ref, lh_ref, out_ref):
        t = jnp.sum(tail_ref[...], axis=(1, 3))
        t300 = t[:, 0:D]
        p = (tcp_ref[...].T + t300[0:B, :]) / lp_ref[...]
        h = (tch_ref[...].T + t300[B:2 * B, :]) / lh_ref[...]
        out_ref[:, 0, :] = p
        out_ref[:, 1, :] = h
        out_ref[:, 2, :] = jnp.abs(p - h)
        out_ref[:, 3, :] = p * h

    out = pl.pallas_call(
        body,
        out_shape=jax.ShapeDtypeStruct((B, 4, D), jnp.float32),
    )(tcp, tch, tails.reshape(PAIRS, NTCH, DPAD, LANES),
      lengths_p.astype(jnp.float32).reshape(B, 1),
      lengths_h.astype(jnp.float32).reshape(B, 1))
    return out.reshape(B, 4 * D)


def kernel(premises, lengths_premises, hypothesis, lengths_hypothesis):
    prem_t = jnp.transpose(premises, (2, 0, 1))
    hyp_t = jnp.transpose(hypothesis, (2, 0, 1))
    lp = lengths_premises.astype(jnp.int32)
    lh = lengths_hypothesis.astype(jnp.int32)
    tails = _sc_tail_sums(prem_t, lp, hyp_t, lh)
    tcp, tch = _tc_dense_sums(prem_t, hyp_t,
                              lp.reshape(1, B), lh.reshape(1, B))
    return _finale(tcp, tch, tails, lengths_premises, lengths_hypothesis)


# speedup vs baseline: 6.6423x; 1.8503x over previous
import dataclasses
import functools

import jax
import jax.numpy as jnp
from jax import lax
from jax.experimental import pallas as pl
from jax.experimental.pallas import tpu as pltpu
from jax.experimental.pallas import tpu_sc as plsc

B, L, D = 16, 2048, 300
NC, NS = 2, 16
NW = NC * NS
LANES = 16
NFG = (D + LANES - 1) // LANES
DPAD = NFG * LANES
PAIRS = 2 * B

KCOLS = 1280
CC = 256
TCR = 128
NTCH = (L - KCOLS) // TCR
CVR = TCR // LANES
ACCW = DPAD + 1


def _tc_dense_sums(prem_t, hyp_t, lens_p2, lens_h2):
    def body(lp_ref, lh_ref, p_ref, h_ref, pout_ref, hout_ref):
        j = pl.program_id(0)
        ci = lax.broadcasted_iota(jnp.int32, (1, B, CC), 2) + j * CC
        mp = (ci < lp_ref[...].reshape(1, B, 1)).astype(jnp.float32)
        mh = (ci < lh_ref[...].reshape(1, B, 1)).astype(jnp.float32)
        ps = jnp.sum(p_ref[...] * mp, axis=2)
        hs = jnp.sum(h_ref[...] * mh, axis=2)

        @pl.when(j == 0)
        def _():
            pout_ref[...] = ps
            hout_ref[...] = hs

        @pl.when(j > 0)
        def _():
            pout_ref[...] += ps
            hout_ref[...] += hs

    return pl.pallas_call(
        body,
        grid=(KCOLS // CC,),
        in_specs=[
            pl.BlockSpec((1, B), lambda j: (0, 0)),
            pl.BlockSpec((1, B), lambda j: (0, 0)),
            pl.BlockSpec((D, B, CC), lambda j: (0, 0, j)),
            pl.BlockSpec((D, B, CC), lambda j: (0, 0, j)),
        ],
        out_specs=[
            pl.BlockSpec((D, B), lambda j: (0, 0)),
            pl.BlockSpec((D, B), lambda j: (0, 0)),
        ],
        out_shape=[jax.ShapeDtypeStruct((D, B), jnp.float32)] * 2,
    )(lens_p2, lens_h2, prem_t, hyp_t)


def _sc_tail_sums(prem_t, lens_p, hyp_t, lens_h):
    mesh = plsc.VectorSubcoreMesh(
        core_axis_name="c", subcore_axis_name="s",
        num_cores=NC, num_subcores=NS)
    cp = pltpu.CompilerParams()
    if "needs_layout_passes" in pltpu.CompilerParams.__dataclass_fields__:
        cp = dataclasses.replace(cp, needs_layout_passes=False)

    @functools.partial(
        pl.kernel,
        compiler_params=cp,
        out_type=jax.ShapeDtypeStruct((PAIRS, NTCH, ACCW * LANES),
                                      jnp.float32),
        mesh=mesh,
        scratch_types=[
            pltpu.VMEM((DPAD, TCR), jnp.float32),
            pltpu.VMEM((DPAD, TCR), jnp.float32),
            pltpu.VMEM((ACCW * LANES,), jnp.float32),
            pltpu.VMEM((ACCW * LANES,), jnp.float32),
            pltpu.VMEM((LANES,), jnp.int32),
            pltpu.VMEM((LANES,), jnp.int32),
            pltpu.SemaphoreType.DMA,
            pltpu.SemaphoreType.DMA,
        ],
    )
    def k(prem_hbm, lenp_hbm, hyp_hbm, lenh_hbm, out_hbm,
          buf0, buf1, acc0, acc1, lpv, lhv, sem0, sem1):
        wid = lax.axis_index("s") * NC + lax.axis_index("c")
        pltpu.sync_copy(lenp_hbm, lpv)
        pltpu.sync_copy(lenh_hbm, lhv)
        iota = lax.iota(jnp.int32, LANES)
        fzero = jnp.zeros((LANES,), jnp.float32)
        scat_base = iota * ACCW

        lpvec = lpv[...]
        lhvec = lhv[...]

        def lane(vec, s):
            return lax.reduce_max(jnp.where(iota == s, vec, 0), axes=(0,))

        def slot_info(d):
            p = lax.rem(wid + NW - d, NW)
            pm = lax.rem(p, B)
            length = jnp.where(p < B, lane(lpvec, pm), lane(lhvec, pm))
            vcols = jnp.clip(length - (KCOLS + d * TCR), 0, TCR)
            return p, pm, vcols

        def start_dma(d, buf, sem):
            p, pm, vcols = slot_info(d)
            c0 = KCOLS + d * TCR

            @pl.when((vcols > 0) & (p < B))
            def _():
                pltpu.async_copy(
                    prem_hbm.at[:, pm, pl.ds(c0, TCR)],
                    buf.at[pl.ds(0, D), :], sem)

            @pl.when((vcols > 0) & (p >= B))
            def _():
                pltpu.async_copy(
                    hyp_hbm.at[:, pm, pl.ds(c0, TCR)],
                    buf.at[pl.ds(0, D), :], sem)

        def consume(d, buf, acc, sem):
            p, pm, vcols = slot_info(d)
            c0 = KCOLS + d * TCR

            @pl.when((vcols > 0) & (p < B))
            def _():
                pltpu.make_async_copy(
                    prem_hbm.at[:, pm, pl.ds(c0, TCR)],
                    buf.at[pl.ds(0, D), :], sem).wait()

            @pl.when((vcols > 0) & (p >= B))
            def _():
                pltpu.make_async_copy(
                    hyp_hbm.at[:, pm, pl.ds(c0, TCR)],
                    buf.at[pl.ds(0, D), :], sem).wait()

            @pl.when(vcols > 0)
            def _():
                masks = [(c * LANES + iota) < vcols for c in range(CVR)]

                @pl.loop(0, DPAD, step=2)
                def _(f0):
                    for f in (f0, f0 + 1):
                        v = fzero
                        for c in range(CVR):
                            x = buf[f, pl.ds(c * LANES, LANES)]
                            v = v + jnp.where(masks[c], x, 0.0)
                        plsc.store_scatter(acc, [scat_base + f], v)

            @pl.when(vcols == 0)
            def _():
                @pl.loop(0, ACCW)
                def _(f):
                    acc[pl.ds(f * LANES, LANES)] = fzero

            pltpu.sync_copy(acc, out_hbm.at[p, d])

        bufs = (buf0, buf1)
        accs = (acc0, acc1)
        sems = (sem0, sem1)
        start_dma(0, bufs[0], sems[0])
        for d in range(NTCH):
            if d + 1 < NTCH:
                start_dma(d + 1, bufs[(d + 1) % 2], sems[(d + 1) % 2])
            consume(d, bufs[d % 2], accs[d % 2], sems[d % 2])

    out = k(prem_t, lens_p, hyp_t, lens_h)
    return out


def _finale(tcp, tch, tails, lengths_p, lengths_h):
    def body(tcp_ref, tch_ref, tail_ref, lp_ref, lh_ref, out_ref):
        t = jnp.sum(tail_ref[...], axis=1)
        tsum = t[:, 0:ACCW]
        for lane in range(1, LANES):
            tsum = tsum + t[:, lane * ACCW:(lane + 1) * ACCW]
        t300 = tsum[:, 0:D]
        p = (tcp_ref[...].T + t300[0:B, :]) / lp_ref[...]
        h = (tch_ref[...].T + t300[B:2 * B, :]) / lh_ref[...]
        out_ref[:, 0, :] = p
        out_ref[:, 1, :] = h
        out_ref[:, 2, :] = jnp.abs(p - h)
        out_ref[:, 3, :] = p * h

    out = pl.pallas_call(
        body,
        out_shape=jax.ShapeDtypeStruct((B, 4, D), jnp.float32),
    )(tcp, tch, tails,
      lengths_p.astype(jnp.float32).reshape(B, 1),
      lengths_h.astype(jnp.float32).reshape(B, 1))
    return out.reshape(B, 4 * D)


def kernel(premises, lengths_premises, hypothesis, lengths_hypothesis):
    prem_t = jnp.transpose(premises, (2, 0, 1))
    hyp_t = jnp.transpose(hypothesis, (2, 0, 1))
    lp = lengths_premises.astype(jnp.int32)
    lh = lengths_hypothesis.astype(jnp.int32)
    tails = _sc_tail_sums(prem_t, lp, hyp_t, lh)
    tcp, tch = _tc_dense_sums(prem_t, hyp_t,
                              lp.reshape(1, B), lh.reshape(1, B))
    return _finale(tcp, tch, tails, lengths_premises, lengths_hypothesis)
